# baseline (speedup 1.0000x reference)
import functools

import jax
import jax.numpy as jnp
from jax import lax
from jax.experimental import pallas as pl
from jax.experimental.pallas import tpu as pltpu
from jax.experimental.pallas import tpu_sc as plsc

_D = 32
_CH = 128


@functools.cache
def _make_lookup(B: int, V: int):
    info = plsc.get_sparse_core_info()
    nc, ns = info.num_cores, info.num_subcores
    nw = nc * ns
    b_per_w = B // nw
    chunks = b_per_w // _CH
    mesh = plsc.VectorSubcoreMesh(core_axis_name="c", subcore_axis_name="s")

    @functools.partial(
        pl.kernel,
        mesh=mesh,
        out_type=jax.ShapeDtypeStruct((_D, B), jnp.float32),
        scratch_types=[
            pltpu.VMEM((chunks, _CH), jnp.int32),
            pltpu.VMEM((_CH, _D), jnp.float32),
            pltpu.VMEM((_CH, _D), jnp.float32),
            pltpu.VMEM((_CH, _D), jnp.float32),
            pltpu.VMEM((_CH, _D), jnp.float32),
            pltpu.VMEM((_D, _CH), jnp.float32),
            pltpu.VMEM((_D, _CH), jnp.float32),
            pltpu.SemaphoreType.DMA,
            pltpu.SemaphoreType.DMA,
            pltpu.SemaphoreType.DMA,
            pltpu.SemaphoreType.DMA,
            pltpu.SemaphoreType.DMA,
            pltpu.SemaphoreType.DMA,
        ],
        compiler_params=pltpu.CompilerParams(
            use_tc_tiling_on_sc=False, needs_layout_passes=False,
            disable_bounds_checks=True),
    )
    def lookup(table_hbm, idx_hbm, out_hbm, idx_v,
               gbuf0, gbuf1, gbuf2, gbuf3, obuf0, obuf1,
               g0, g1, g2, g3, w0, w1):
        c = lax.axis_index("c")
        s = lax.axis_index("s")
        wid = s * nc + c
        i32 = jnp.int32
        iota = lax.iota(i32, 16)
        rows2 = [iota, iota + 16]

        pltpu.sync_copy(idx_hbm.at[wid], idx_v)
        gbufs, obufs = (gbuf0, gbuf1, gbuf2, gbuf3), (obuf0, obuf1)
        gsems, wsems = (g0, g1, g2, g3), (w0, w1)

        def b_fire(j, par):
            pltpu.async_copy(table_hbm.at[idx_v.at[j]], gbufs[par], gsems[par])

        def b_wait_g(par):
            pltpu.make_async_copy(
                table_hbm.at[pl.ds(0, _CH)], gbufs[par], gsems[par]).wait()

        cols_l = [jnp.full((16,), l, dtype=i32) for l in range(_CH)]

        def b_transpose(gbuf, obuf):
            for lg in range(16):
                vecs = []
                for li in range(8):
                    l = lg * 8 + li
                    for dd in range(2):
                        vecs.append((l, dd, gbuf[l, pl.ds(dd * 16, 16)]))
                for l, dd, v in vecs:
                    plsc.store_scatter(obuf, [rows2[dd], cols_l[l]], v)

        def b_write(j, par):
            pltpu.async_copy(
                obufs[par],
                out_hbm.at[:, pl.ds((wid * chunks + j) * _CH, _CH)],
                wsems[par])

        def b_drain_w(par):
            pltpu.make_async_copy(
                obufs[par], out_hbm.at[:, pl.ds(0, _CH)], wsems[par]).wait()

        b_fire(0, 0)
        b_fire(1, 1)
        b_fire(2, 2)

        def b_proc(j, gpar, opar):
            b_wait_g(gpar)

            @pl.when(j >= 2)
            def _():
                b_drain_w(opar)

            b_transpose(gbufs[gpar], obufs[opar])
            b_write(j, opar)

        def b_body(t, carry):
            for k in range(4):
                j = 4 * t + k

                @pl.when(j + 3 < chunks)
                def _():
                    b_fire(j + 3, (k + 3) % 4)

                b_proc(j, k, k % 2)
            return carry

        lax.fori_loop(0, chunks // 4, b_body, 0)
        b_drain_w(0)
        b_drain_w(1)

    return lookup


def kernel(x, weight):
    B = x.shape[0] * x.shape[1]
    info = plsc.get_sparse_core_info()
    nw = info.num_cores * info.num_subcores
    idx = x.reshape(nw, (B // nw) // _CH, _CH)
    out_t = _make_lookup(B, weight.shape[0])(weight, idx)
    return out_t.T.reshape(x.shape[0], x.shape[1], _D)


# speedup vs baseline: 1.6857x; 1.6857x over previous
import functools

import jax
import jax.numpy as jnp
from jax import lax
from jax.experimental import pallas as pl
from jax.experimental.pallas import tpu as pltpu
from jax.experimental.pallas import tpu_sc as plsc

_D = 32
_CH = 128


@functools.cache
def _make_lookup(B: int, V: int):
    info = plsc.get_sparse_core_info()
    nc, ns = info.num_cores, info.num_subcores
    nw = nc * ns
    b_per_w = B // nw
    chunks = b_per_w // _CH
    mesh = plsc.VectorSubcoreMesh(core_axis_name="c", subcore_axis_name="s")

    @functools.partial(
        pl.kernel,
        mesh=mesh,
        out_type=jax.ShapeDtypeStruct((B, _D), jnp.float32),
        scratch_types=[
            pltpu.VMEM((chunks, _CH), jnp.int32),
            pltpu.VMEM((_CH, _D), jnp.float32),
            pltpu.VMEM((_CH, _D), jnp.float32),
            pltpu.VMEM((_CH, _D), jnp.float32),
            pltpu.VMEM((_CH, _D), jnp.float32),
            pltpu.SemaphoreType.DMA,
            pltpu.SemaphoreType.DMA,
            pltpu.SemaphoreType.DMA,
            pltpu.SemaphoreType.DMA,
        ],
        compiler_params=pltpu.CompilerParams(
            use_tc_tiling_on_sc=False, disable_bounds_checks=True),
    )
    def lookup(table_hbm, idx_hbm, out_hbm, idx_v,
               gbuf0, gbuf1, gbuf2, gbuf3, g0, g1, g2, g3):
        c = lax.axis_index("c")
        s = lax.axis_index("s")
        wid = s * nc + c
        base = wid * b_per_w

        pltpu.sync_copy(idx_hbm.at[wid], idx_v)
        gbufs = (gbuf0, gbuf1, gbuf2, gbuf3)
        gsems = (g0, g1, g2, g3)

        def b_fire(j, par):
            pltpu.async_copy(table_hbm.at[idx_v.at[j]], gbufs[par], gsems[par])

        def b_wait_g(par):
            pltpu.make_async_copy(
                table_hbm.at[pl.ds(0, _CH)], gbufs[par], gsems[par]).wait()

        def b_write(j, par):
            pltpu.sync_copy(
                gbufs[par], out_hbm.at[pl.ds(base + j * _CH, _CH)])

        b_fire(0, 0)
        b_fire(1, 1)
        b_fire(2, 2)

        def b_body(t, carry):
            for k in range(4):
                j = 4 * t + k

                @pl.when(j + 3 < chunks)
                def _():
                    b_fire(j + 3, (k + 3) % 4)

                b_wait_g(k)
                b_write(j, k)
            return carry

        lax.fori_loop(0, chunks // 4, b_body, 0)

    return lookup


def kernel(x, weight):
    B = x.shape[0] * x.shape[1]
    info = plsc.get_sparse_core_info()
    nw = info.num_cores * info.num_subcores
    idx = x.reshape(nw, (B // nw) // _CH, _CH)
    out = _make_lookup(B, weight.shape[0])(weight, idx)
    return out.reshape(x.shape[0], x.shape[1], _D)
